# baseline (speedup 1.0000x reference)
import jax
import jax.numpy as jnp
from jax.experimental import pallas as pl
from jax.experimental.pallas import tpu as pltpu

_N = 10000
_F = 128
_R = 200


def _sage_body(adj_ref, x_ref, w_ref, b_ref, out_ref):
    a = adj_ref[...]
    deg = jnp.maximum(jnp.sum(a, axis=1, keepdims=True), 1.0)
    acc = jnp.dot(a, x_ref[...], preferred_element_type=jnp.float32)
    agg = acc / deg
    xs = x_ref[pl.ds(pl.program_id(0) * _R, _R), :]
    dn = (((1,), (1,)), ((), ()))
    h = (
        jax.lax.dot_general(xs, w_ref[:, :_F], dn,
                            preferred_element_type=jnp.float32)
        + jax.lax.dot_general(agg, w_ref[:, _F:], dn,
                              preferred_element_type=jnp.float32)
        + b_ref[...]
    )
    out_ref[...] = jnp.maximum(h, 0.0)


def kernel(x, adj, W, b):
    return pl.pallas_call(
        _sage_body,
        grid=(_N // _R,),
        in_specs=[
            pl.BlockSpec((_R, _N), lambda i: (i, 0)),
            pl.BlockSpec((_N, _F), lambda i: (0, 0)),
            pl.BlockSpec((_F, 2 * _F), lambda i: (0, 0)),
            pl.BlockSpec((1, _F), lambda i: (0, 0)),
        ],
        out_specs=pl.BlockSpec((_R, _F), lambda i: (i, 0)),
        out_shape=jax.ShapeDtypeStruct((_N, _F), jnp.float32),
        compiler_params=pltpu.CompilerParams(
            dimension_semantics=("parallel",),
        ),
    )(adj, x, W, b[None, :])


# speedup vs baseline: 1.0566x; 1.0566x over previous
import jax
import jax.numpy as jnp
from jax.experimental import pallas as pl
from jax.experimental.pallas import tpu as pltpu

_N = 10000
_F = 128
_R = 400


def _sage_body(adj_ref, x_ref, w_ref, b_ref, out_ref):
    a = adj_ref[...]
    deg = jnp.maximum(jnp.sum(a, axis=1, keepdims=True), 1.0)
    acc = jnp.dot(a, x_ref[...], preferred_element_type=jnp.float32)
    agg = acc / deg
    xs = x_ref[pl.ds(pl.program_id(0) * _R, _R), :]
    dn = (((1,), (1,)), ((), ()))
    h = (
        jax.lax.dot_general(xs, w_ref[:, :_F], dn,
                            preferred_element_type=jnp.float32)
        + jax.lax.dot_general(agg, w_ref[:, _F:], dn,
                              preferred_element_type=jnp.float32)
        + b_ref[...]
    )
    out_ref[...] = jnp.maximum(h, 0.0)


def kernel(x, adj, W, b):
    return pl.pallas_call(
        _sage_body,
        grid=(_N // _R,),
        in_specs=[
            pl.BlockSpec((_R, _N), lambda i: (i, 0)),
            pl.BlockSpec((_N, _F), lambda i: (0, 0)),
            pl.BlockSpec((_F, 2 * _F), lambda i: (0, 0)),
            pl.BlockSpec((1, _F), lambda i: (0, 0)),
        ],
        out_specs=pl.BlockSpec((_R, _F), lambda i: (i, 0)),
        out_shape=jax.ShapeDtypeStruct((_N, _F), jnp.float32),
        compiler_params=pltpu.CompilerParams(
            dimension_semantics=("parallel",),
        ),
    )(adj, x, W, b[None, :])
